# baseline (speedup 1.0000x reference)
import numpy as np
import jax
import jax.numpy as jnp
from jax.experimental import pallas as pl

_P = 1.0
_FREQ_MASK_PARAM = 27
_TIME_MASK_PARAM = 100
_FREQ_MASKS = 2
_TIME_MASKS = 2


def _mask_vectors(batch, n_freq, n_time):
    rng = np.random.default_rng(0)
    if rng.random() > _P:
        return None
    rowm = np.zeros((batch, n_freq), np.float32)
    colm = np.zeros((batch, n_time), np.float32)
    for idx in range(batch):
        for _ in range(_FREQ_MASKS):
            max_w = min(_FREQ_MASK_PARAM, n_freq)
            w = int(rng.integers(0, max_w + 1))
            if w > 0:
                s = int(rng.integers(0, n_freq - w + 1))
                rowm[idx, s:s + w] = 1.0
        for _ in range(_TIME_MASKS):
            max_w = min(_TIME_MASK_PARAM, n_time)
            w = int(rng.integers(0, max_w + 1))
            if w > 0:
                s = int(rng.integers(0, n_time - w + 1))
                colm[idx, s:s + w] = 1.0
    return rowm, colm


def _body(x_ref, rowm_ref, colm_ref, o_ref):
    xb = x_ref[0]
    fill = jnp.mean(xb)
    rm = rowm_ref[0, 0, :]
    cm = colm_ref[0, 0, :]
    m = jnp.maximum(rm[:, None], cm[None, :]) > 0.0
    o_ref[0] = jnp.where(m, fill, xb)


def kernel(x):
    batch, ch, n_freq, n_time = x.shape
    masks = _mask_vectors(batch, n_freq, n_time)
    if masks is None:
        return x
    rowm_np, colm_np = masks
    rowm = jnp.asarray(rowm_np).reshape(batch, 1, n_freq)
    colm = jnp.asarray(colm_np).reshape(batch, 1, n_time)
    x3 = x.reshape(batch * ch, n_freq, n_time)

    out = pl.pallas_call(
        _body,
        grid=(batch,),
        in_specs=[
            pl.BlockSpec((1, n_freq, n_time), lambda b: (b, 0, 0)),
            pl.BlockSpec((1, 1, n_freq), lambda b: (b, 0, 0)),
            pl.BlockSpec((1, 1, n_time), lambda b: (b, 0, 0)),
        ],
        out_specs=pl.BlockSpec((1, n_freq, n_time), lambda b: (b, 0, 0)),
        out_shape=jax.ShapeDtypeStruct((batch * ch, n_freq, n_time), x.dtype),
    )(x3, rowm, colm)
    return out.reshape(batch, ch, n_freq, n_time)
